# Initial kernel scaffold; baseline (speedup 1.0000x reference)
import jax
import jax.numpy as jnp
from jax.experimental import pallas as pl


def kernel(x, table, bias):
    raise NotImplementedError("write your pallas kernel here")


# speedup vs baseline: 1.4494x; 1.4494x over previous
import functools

import jax
import jax.numpy as jnp
import numpy as np
from jax import lax
from jax.experimental import pallas as pl
from jax.experimental.pallas import tpu as pltpu
from jax.experimental.pallas import tpu_sc as plsc

_FIELD_DIMS = [40000] * 26
_OFFSETS = np.array((0, *np.cumsum(_FIELD_DIMS)[:-1]), dtype=np.int32)

_B = 16384
_F = 26
_NC = 2
_NS = 16
_NW = _NC * _NS
_BPW = _B // _NW
_L = 16


def _make_sc_kernel():
  mesh = plsc.VectorSubcoreMesh(
      core_axis_name="c", subcore_axis_name="s",
      num_cores=_NC, num_subcores=_NS)

  @functools.partial(
      pl.kernel,
      mesh=mesh,
      out_type=jax.ShapeDtypeStruct((_B,), jnp.float32),
      scratch_types=[
          pltpu.VMEM((_F * _BPW,), jnp.int32),
          pltpu.VMEM((_F * _BPW,), jnp.float32),
          pltpu.VMEM((_BPW,), jnp.float32),
          pltpu.SemaphoreType.DMA,
      ],
  )
  def sc_kernel(idx_hbm, table_hbm, out_hbm, idx_v, vals_v, acc_v, sem):
    wid = lax.axis_index("s") * _NC + lax.axis_index("c")
    pltpu.sync_copy(idx_hbm.at[wid], idx_v)
    pltpu.async_copy(table_hbm.at[idx_v], vals_v, sem).wait()
    for rc in range(_BPW // _L):
      acc = vals_v[pl.ds(rc * _L, _L)]
      for f in range(1, _F):
        acc = acc + vals_v[pl.ds(f * _BPW + rc * _L, _L)]
      acc_v[pl.ds(rc * _L, _L)] = acc
    pltpu.sync_copy(acc_v, out_hbm.at[pl.ds(wid * _BPW, _BPW)])

  return sc_kernel


_SC_KERNEL = _make_sc_kernel()


def kernel(x, table, bias):
  offsets = jnp.asarray(_OFFSETS)
  idx = x.astype(jnp.int32) + offsets[None, :]
  idx = idx.reshape(_NW, _BPW, _F).transpose(0, 2, 1).reshape(_NW, _F * _BPW)
  sums = _SC_KERNEL(idx, table.reshape(-1))
  return sums[:, None] + bias[None, :]


# speedup vs baseline: 2.5160x; 1.7358x over previous
import functools

import jax
import jax.numpy as jnp
import numpy as np
from jax import lax
from jax.experimental import pallas as pl
from jax.experimental.pallas import tpu as pltpu
from jax.experimental.pallas import tpu_sc as plsc

_FIELD_DIMS = [40000] * 26
_OFFSETS = np.array((0, *np.cumsum(_FIELD_DIMS)[:-1]), dtype=np.int32)

_B = 16384
_F = 26
_V = sum(_FIELD_DIMS)
_VPAD = (_V + 1023) // 1024 * 1024
_NC = 2
_NS = 16
_NW = _NC * _NS
_BPW = _B // _NW
_L = 16


def _make_sc_kernel():
  mesh = plsc.VectorSubcoreMesh(
      core_axis_name="c", subcore_axis_name="s",
      num_cores=_NC, num_subcores=_NS)

  @functools.partial(
      pl.kernel,
      mesh=mesh,
      out_type=jax.ShapeDtypeStruct((_B,), jnp.float32),
      scratch_types=[
          pltpu.VMEM((_F * _BPW,), jnp.int32),
          pltpu.VMEM((_F * _BPW,), jnp.float32),
          pltpu.VMEM((_BPW,), jnp.float32),
          pltpu.SemaphoreType.DMA,
      ],
  )
  def sc_kernel(idx_hbm, table_hbm, out_hbm, idx_v, vals_v, acc_v, sem):
    wid = lax.axis_index("s") * _NC + lax.axis_index("c")
    pltpu.sync_copy(idx_hbm.at[wid], idx_v)
    pltpu.async_copy(table_hbm.at[idx_v], vals_v, sem).wait()
    for rc in range(_BPW // _L):
      acc = vals_v[pl.ds(rc * _L, _L)]
      for f in range(1, _F):
        acc = acc + vals_v[pl.ds(f * _BPW + rc * _L, _L)]
      acc_v[pl.ds(rc * _L, _L)] = acc
    pltpu.sync_copy(acc_v, out_hbm.at[pl.ds(wid * _BPW, _BPW)])

  return sc_kernel


_SC_KERNEL = _make_sc_kernel()


def kernel(x, table, bias):
  offsets = jnp.asarray(_OFFSETS)
  idx = x.astype(jnp.int32) + offsets[None, :]
  idx = idx.reshape(_NW, _BPW, _F).transpose(0, 2, 1).reshape(_NW, _F * _BPW)
  tpad = lax.pad(table, jnp.float32(0), ((0, _VPAD - _V, 0), (0, 0, 0)))
  sums = _SC_KERNEL(idx, tpad.reshape(-1))
  return sums[:, None] + bias[None, :]
